# baseline (speedup 1.0000x reference)
import functools

import jax
import jax.numpy as jnp
import numpy as np
from jax import lax
from jax.experimental import pallas as pl
from jax.experimental.pallas import tpu as pltpu
from jax.experimental.pallas import tpu_sc as plsc

_NL = 16
_T = 1 << 19
_PRIME_I32 = -1640531535
_MASK = _T - 1
_LVL = [0, 1, 8, 9, 2, 3, 10, 11, 4, 5, 12, 13, 6, 7, 14, 15]
_ROW_PERM = [0, 2, 4, 6, 1, 3, 5, 7]

_BPA = 2048
_BPC = 1024
_GW = 2048
_GR = 8


def _idx_body(pos_ref, out_ref):
    px = pos_ref[0:1, :]
    py = pos_ref[1:2, :]
    rows = []
    for lvl in _LVL:
        res = float(1 << (4 + lvl))
        x0 = jnp.floor(px * res).astype(jnp.int32)
        y0 = jnp.floor(py * res).astype(jnp.int32)
        base = jnp.int32(lvl * _T)
        for dx in (0, 1):
            cx = x0 + jnp.int32(dx)
            for dy in (0, 1):
                cy = y0 + jnp.int32(dy)
                h = (cx ^ (cy * jnp.int32(_PRIME_I32))) & jnp.int32(_MASK)
                rows.append(h + base)
    out_ref[...] = jnp.concatenate(rows, axis=0)


def _compute_idx(pos_t):
    b = pos_t.shape[1]
    return pl.pallas_call(
        _idx_body,
        grid=(b // _BPA,),
        in_specs=[pl.BlockSpec((2, _BPA), lambda i: (0, i))],
        out_specs=pl.BlockSpec((64, _BPA), lambda i: (0, i)),
        out_shape=jax.ShapeDtypeStruct((64, b), jnp.int32),
    )(pos_t)


def _sc_gather(table_i32, idx_rows):
    nrows = idx_rows.shape[0]
    mesh = plsc.VectorSubcoreMesh(
        core_axis_name="core", subcore_axis_name="subcore")

    @functools.partial(
        pl.kernel,
        out_type=jax.ShapeDtypeStruct((nrows, _GW), jnp.int32),
        mesh=mesh,
        scratch_types=[pltpu.SemaphoreType.DMA],
        compiler_params=pltpu.CompilerParams(use_tc_tiling_on_sc=False),
    )
    def k(table_hbm, idx_hbm, out_hbm, sem):
        def body(i_vmem, o_vmem):
            handles = [
                pltpu.async_copy(
                    table_hbm.at[i_vmem.at[j]], o_vmem.at[j], sem)
                for j in range(_GR)
            ]
            for h in handles:
                h.wait()

        pltpu.emit_pipeline(
            body,
            grid=(nrows // _GR,),
            in_specs=[pl.BlockSpec((_GR, _GW), lambda i: (i, 0))],
            out_specs=[pl.BlockSpec((_GR, _GW), lambda i: (i, 0))],
            core_axis_name=("core", "subcore"),
            dimension_semantics=(pltpu.PARALLEL,),
        )(idx_hbm, out_hbm)

    return k(table_i32, idx_rows)


def _unpack_lo(x):
    return lax.bitcast_convert_type(jnp.left_shift(x, 16), jnp.float32)


def _unpack_hi(x):
    return lax.bitcast_convert_type(
        jnp.bitwise_and(x, jnp.int32(-65536)), jnp.float32)


def _mlp_body(pos_ref, g_ref,
              w0t0, w0t1, w0t2, w0t3,
              w1t0, w1t1, w1t2, w1t3,
              w2t0, w2t1, w2t2, w2t3,
              out_ref):
    px = pos_ref[0:1, :]
    py = pos_ref[1:2, :]
    acc0 = []
    acc1 = []
    for g, lvl in enumerate(_LVL):
        res = float(1 << (4 + lvl))
        sx = px * res
        sy = py * res
        wx1 = sx - jnp.floor(sx)
        wy1 = sy - jnp.floor(sy)
        wx0 = 1.0 - wx1
        wy0 = 1.0 - wy1
        ws = (wx0 * wy0, wx0 * wy1, wx1 * wy0, wx1 * wy1)
        packed = [g_ref[4 * g + c:4 * g + c + 1, :] for c in range(4)]
        a0 = _unpack_lo(packed[0]) * ws[0]
        a1 = _unpack_hi(packed[0]) * ws[0]
        for c in (1, 2, 3):
            a0 = a0 + _unpack_lo(packed[c]) * ws[c]
            a1 = a1 + _unpack_hi(packed[c]) * ws[c]
        acc0.append(a0)
        acc1.append(a1)
    parts = []
    for i in range(4):
        parts.extend(acc0[4 * i:4 * i + 4])
        parts.extend(acc1[4 * i:4 * i + 4])
    feats = jnp.concatenate(parts, axis=0)
    w0ts = [w0t0, w0t1, w0t2, w0t3]
    w1ts = [w1t0, w1t1, w1t2, w1t3]
    w2ts = [w2t0, w2t1, w2t2, w2t3]
    outs = []
    for i in range(4):
        k = 8 * (i + 1)
        a = feats[:k, :]
        h = jnp.maximum(
            jnp.dot(w0ts[i][...], a, preferred_element_type=jnp.float32,
                    precision=lax.Precision.HIGHEST), 0.0)
        h = jnp.maximum(
            jnp.dot(w1ts[i][...], h, preferred_element_type=jnp.float32,
                    precision=lax.Precision.HIGHEST), 0.0)
        outs.append(
            jnp.dot(w2ts[i][...], h, preferred_element_type=jnp.float32,
                    precision=lax.Precision.HIGHEST))
    out_ref[...] = jnp.concatenate(outs, axis=0)


def _mlp(pos_t, g, w0ts, w1ts, w2ts):
    b = pos_t.shape[1]
    full = lambda arr: pl.BlockSpec(arr.shape, lambda i: (0, 0))
    in_specs = [
        pl.BlockSpec((2, _BPC), lambda i: (0, i)),
        pl.BlockSpec((64, _BPC), lambda i: (0, i)),
    ] + [full(w) for w in w0ts + w1ts + w2ts]
    return pl.pallas_call(
        _mlp_body,
        grid=(b // _BPC,),
        in_specs=in_specs,
        out_specs=pl.BlockSpec((4, _BPC), lambda i: (0, i)),
        out_shape=jax.ShapeDtypeStruct((4, b), jnp.float32),
    )(pos_t, g, *w0ts, *w1ts, *w2ts)


def _prep_weights(mlp_params):
    w0ts, w1ts, w2ts = [], [], []
    for i in range(4):
        w0, w1, w2 = mlp_params[i]
        prows = []
        for j in range(i + 1):
            prows.extend(8 * j + m for m in _ROW_PERM)
        w0ts.append(w0[np.array(prows), :].T)
        w1ts.append(w1.T)
        w2ts.append(w2.T)
    return w0ts, w1ts, w2ts


def kernel(v_pixel_pos, hash_table, mlp_params):
    b = v_pixel_pos.shape[0]
    pos_t = v_pixel_pos.T
    idx = _compute_idx(pos_t)
    table_i32 = lax.bitcast_convert_type(
        hash_table.astype(jnp.bfloat16), jnp.int32).reshape(_NL * _T)
    gathered = _sc_gather(table_i32, idx.reshape(64 * b // _GW, _GW))
    g = gathered.reshape(64, b)
    w0ts, w1ts, w2ts = _prep_weights(mlp_params)
    out4 = _mlp(pos_t, g, w0ts, w1ts, w2ts)
    return tuple(out4[i].reshape(b, 1) for i in range(4))


# speedup vs baseline: 1.1575x; 1.1575x over previous
import functools

import jax
import jax.numpy as jnp
import numpy as np
from jax import lax
from jax.experimental import pallas as pl
from jax.experimental.pallas import tpu as pltpu
from jax.experimental.pallas import tpu_sc as plsc

_NL = 16
_T = 1 << 19
_PRIME_I32 = -1640531535
_MASK = _T - 1
_LVL = [0, 1, 8, 9, 2, 3, 10, 11, 4, 5, 12, 13, 6, 7, 14, 15]
_ROW_PERM = [0, 2, 4, 6, 1, 3, 5, 7]

_BP = 2048
_GR = 8

_RES_F32 = [float(1 << (4 + lvl)) for lvl in _LVL]
_BASE_I32 = [lvl * _T for lvl in _LVL]


def _idx_body(res_ref, base_ref, pos_ref, out_ref):
    px = pos_ref[0:1, :]
    py = pos_ref[1:2, :]
    res = res_ref[...]
    base = base_ref[...]
    sx = res * px
    sy = res * py
    x0 = jnp.floor(sx).astype(jnp.int32)
    y0 = jnp.floor(sy).astype(jnp.int32)
    rows = []
    for dx in (0, 1):
        cx = x0 + jnp.int32(dx)
        for dy in (0, 1):
            cy = y0 + jnp.int32(dy)
            h = (cx ^ (cy * jnp.int32(_PRIME_I32))) & jnp.int32(_MASK)
            rows.append(h + base)
    out_ref[...] = jnp.concatenate(rows, axis=0)


def _compute_idx(pos_t):
    b = pos_t.shape[1]
    m = b // _BP
    res = jnp.asarray(_RES_F32, jnp.float32).reshape(16, 1)
    base = jnp.asarray(_BASE_I32, jnp.int32).reshape(16, 1)
    return pl.pallas_call(
        _idx_body,
        grid=(m,),
        in_specs=[
            pl.BlockSpec((16, 1), lambda i: (0, 0)),
            pl.BlockSpec((16, 1), lambda i: (0, 0)),
            pl.BlockSpec((2, _BP), lambda i: (0, i)),
        ],
        out_specs=pl.BlockSpec((64, _BP), lambda i: (i, 0)),
        out_shape=jax.ShapeDtypeStruct((m * 64, _BP), jnp.int32),
    )(res, base, pos_t)


def _sc_gather(table_i32, idx2):
    nrows = idx2.shape[0]
    mesh = plsc.VectorSubcoreMesh(
        core_axis_name="core", subcore_axis_name="subcore")

    @functools.partial(
        pl.kernel,
        out_type=jax.ShapeDtypeStruct((nrows, _BP), jnp.int32),
        mesh=mesh,
        scratch_types=[pltpu.SemaphoreType.DMA],
        compiler_params=pltpu.CompilerParams(use_tc_tiling_on_sc=False),
    )
    def k(table_hbm, idx_hbm, out_hbm, sem):
        def body(i_vmem, o_vmem):
            handles = [
                pltpu.async_copy(
                    table_hbm.at[i_vmem.at[j]], o_vmem.at[j], sem)
                for j in range(_GR)
            ]
            for h in handles:
                h.wait()

        pltpu.emit_pipeline(
            body,
            grid=(nrows // _GR,),
            in_specs=[pl.BlockSpec((_GR, _BP), lambda i: (i, 0))],
            out_specs=[pl.BlockSpec((_GR, _BP), lambda i: (i, 0))],
            core_axis_name=("core", "subcore"),
            dimension_semantics=(pltpu.PARALLEL,),
        )(idx_hbm, out_hbm)

    return k(table_i32, idx2)


def _unpack_lo(x):
    return lax.bitcast_convert_type(jnp.left_shift(x, 16), jnp.float32)


def _unpack_hi(x):
    return lax.bitcast_convert_type(
        jnp.bitwise_and(x, jnp.int32(-65536)), jnp.float32)


def _mlp_body(res_ref, pos_ref, g_ref,
              w0t0, w0t1, w0t2, w0t3,
              w1t0, w1t1, w1t2, w1t3,
              w2t0, w2t1, w2t2, w2t3,
              out_ref):
    px = pos_ref[0:1, :]
    py = pos_ref[1:2, :]
    res = res_ref[...]
    sx = res * px
    sy = res * py
    wx1 = sx - jnp.floor(sx)
    wy1 = sy - jnp.floor(sy)
    wx0 = 1.0 - wx1
    wy0 = 1.0 - wy1
    wc = (wx0 * wy0, wx0 * wy1, wx1 * wy0, wx1 * wy1)
    f0 = None
    f1 = None
    for c in range(4):
        p = g_ref[16 * c:16 * c + 16, :]
        a = _unpack_lo(p) * wc[c]
        b = _unpack_hi(p) * wc[c]
        f0 = a if f0 is None else f0 + a
        f1 = b if f1 is None else f1 + b
    parts = []
    for i in range(4):
        parts.append(f0[4 * i:4 * i + 4, :])
        parts.append(f1[4 * i:4 * i + 4, :])
    feats = jnp.concatenate(parts, axis=0)
    w0ts = [w0t0, w0t1, w0t2, w0t3]
    w1ts = [w1t0, w1t1, w1t2, w1t3]
    w2ts = [w2t0, w2t1, w2t2, w2t3]
    outs = []
    for i in range(4):
        k = 8 * (i + 1)
        a = feats[:k, :]
        h = jnp.maximum(
            jnp.dot(w0ts[i][...], a, preferred_element_type=jnp.float32,
                    precision=lax.Precision.HIGHEST), 0.0)
        h = jnp.maximum(
            jnp.dot(w1ts[i][...], h, preferred_element_type=jnp.float32,
                    precision=lax.Precision.HIGHEST), 0.0)
        outs.append(
            jnp.dot(w2ts[i][...], h, preferred_element_type=jnp.float32,
                    precision=lax.Precision.HIGHEST))
    out_ref[...] = jnp.concatenate(outs, axis=0)


def _mlp(pos_t, g3, w0ts, w1ts, w2ts):
    b = pos_t.shape[1]
    m = b // _BP
    res = jnp.asarray(_RES_F32, jnp.float32).reshape(16, 1)
    full = lambda arr: pl.BlockSpec(arr.shape, lambda i: (0, 0))
    in_specs = [
        pl.BlockSpec((16, 1), lambda i: (0, 0)),
        pl.BlockSpec((2, _BP), lambda i: (0, i)),
        pl.BlockSpec((64, _BP), lambda i: (i, 0)),
    ] + [full(w) for w in w0ts + w1ts + w2ts]
    return pl.pallas_call(
        _mlp_body,
        grid=(m,),
        in_specs=in_specs,
        out_specs=pl.BlockSpec((4, _BP), lambda i: (0, i)),
        out_shape=jax.ShapeDtypeStruct((4, b), jnp.float32),
    )(res, pos_t, g3, *w0ts, *w1ts, *w2ts)


def _prep_weights(mlp_params):
    w0ts, w1ts, w2ts = [], [], []
    for i in range(4):
        w0, w1, w2 = mlp_params[i]
        prows = []
        for j in range(i + 1):
            prows.extend(8 * j + m for m in _ROW_PERM)
        w0ts.append(w0[np.array(prows), :].T)
        w1ts.append(w1.T)
        w2ts.append(w2.T)
    return w0ts, w1ts, w2ts


def kernel(v_pixel_pos, hash_table, mlp_params):
    b = v_pixel_pos.shape[0]
    pos_t = v_pixel_pos.T
    idx2 = _compute_idx(pos_t)
    table_i32 = lax.bitcast_convert_type(
        hash_table.astype(jnp.bfloat16), jnp.int32).reshape(_NL * _T)
    g3 = _sc_gather(table_i32, idx2)
    w0ts, w1ts, w2ts = _prep_weights(mlp_params)
    out4 = _mlp(pos_t, g3, w0ts, w1ts, w2ts)
    return tuple(out4[i].reshape(b, 1) for i in range(4))


# speedup vs baseline: 1.1608x; 1.0028x over previous
import functools

import jax
import jax.numpy as jnp
import numpy as np
from jax import lax
from jax.experimental import pallas as pl
from jax.experimental.pallas import tpu as pltpu
from jax.experimental.pallas import tpu_sc as plsc

_NL = 16
_T = 1 << 19
_PRIME_I32 = -1640531535
_MASK = _T - 1
_LVL = [0, 1, 8, 9, 2, 3, 10, 11, 4, 5, 12, 13, 6, 7, 14, 15]
_ROW_PERM = [0, 2, 4, 6, 1, 3, 5, 7]

_BP = 2048
_GR = 16

_RES_F32 = [float(1 << (4 + lvl)) for lvl in _LVL]
_BASE_I32 = [lvl * _T for lvl in _LVL]


def _idx_body(res_ref, base_ref, pos_ref, out_ref):
    px = pos_ref[0:1, :]
    py = pos_ref[1:2, :]
    res = res_ref[...]
    base = base_ref[...]
    sx = res * px
    sy = res * py
    x0 = jnp.floor(sx).astype(jnp.int32)
    y0 = jnp.floor(sy).astype(jnp.int32)
    rows = []
    for dx in (0, 1):
        cx = x0 + jnp.int32(dx)
        for dy in (0, 1):
            cy = y0 + jnp.int32(dy)
            h = (cx ^ (cy * jnp.int32(_PRIME_I32))) & jnp.int32(_MASK)
            rows.append(h + base)
    out_ref[...] = jnp.concatenate(rows, axis=0)


def _compute_idx(pos_t):
    b = pos_t.shape[1]
    m = b // _BP
    res = jnp.asarray(_RES_F32, jnp.float32).reshape(16, 1)
    base = jnp.asarray(_BASE_I32, jnp.int32).reshape(16, 1)
    return pl.pallas_call(
        _idx_body,
        grid=(m,),
        in_specs=[
            pl.BlockSpec((16, 1), lambda i: (0, 0)),
            pl.BlockSpec((16, 1), lambda i: (0, 0)),
            pl.BlockSpec((2, _BP), lambda i: (0, i)),
        ],
        out_specs=pl.BlockSpec((64, _BP), lambda i: (i, 0)),
        out_shape=jax.ShapeDtypeStruct((m * 64, _BP), jnp.int32),
    )(res, base, pos_t)


def _sc_gather(table_i32, idx2):
    nrows = idx2.shape[0]
    mesh = plsc.VectorSubcoreMesh(
        core_axis_name="core", subcore_axis_name="subcore")

    @functools.partial(
        pl.kernel,
        out_type=jax.ShapeDtypeStruct((nrows, _BP), jnp.int32),
        mesh=mesh,
        scratch_types=[pltpu.SemaphoreType.DMA],
        compiler_params=pltpu.CompilerParams(use_tc_tiling_on_sc=False),
    )
    def k(table_hbm, idx_hbm, out_hbm, sem):
        def body(i_vmem, o_vmem):
            handles = [
                pltpu.async_copy(
                    table_hbm.at[i_vmem.at[j]], o_vmem.at[j], sem)
                for j in range(_GR)
            ]
            for h in handles:
                h.wait()

        pltpu.emit_pipeline(
            body,
            grid=(nrows // _GR,),
            in_specs=[pl.BlockSpec((_GR, _BP), lambda i: (i, 0))],
            out_specs=[pl.BlockSpec((_GR, _BP), lambda i: (i, 0))],
            core_axis_name=("core", "subcore"),
            dimension_semantics=(pltpu.PARALLEL,),
        )(idx_hbm, out_hbm)

    return k(table_i32, idx2)


def _unpack_lo(x):
    return lax.bitcast_convert_type(jnp.left_shift(x, 16), jnp.float32)


def _unpack_hi(x):
    return lax.bitcast_convert_type(
        jnp.bitwise_and(x, jnp.int32(-65536)), jnp.float32)


def _mlp_body(res_ref, pos_ref, g_ref,
              w0t0, w0t1, w0t2, w0t3,
              w1t0, w1t1, w1t2, w1t3,
              w2t0, w2t1, w2t2, w2t3,
              out_ref):
    px = pos_ref[0:1, :]
    py = pos_ref[1:2, :]
    res = res_ref[...]
    sx = res * px
    sy = res * py
    wx1 = sx - jnp.floor(sx)
    wy1 = sy - jnp.floor(sy)
    wx0 = 1.0 - wx1
    wy0 = 1.0 - wy1
    wc = (wx0 * wy0, wx0 * wy1, wx1 * wy0, wx1 * wy1)
    f0 = None
    f1 = None
    for c in range(4):
        p = g_ref[16 * c:16 * c + 16, :]
        a = _unpack_lo(p) * wc[c]
        b = _unpack_hi(p) * wc[c]
        f0 = a if f0 is None else f0 + a
        f1 = b if f1 is None else f1 + b
    parts = []
    for i in range(4):
        parts.append(f0[4 * i:4 * i + 4, :])
        parts.append(f1[4 * i:4 * i + 4, :])
    feats = jnp.concatenate(parts, axis=0)
    w0ts = [w0t0, w0t1, w0t2, w0t3]
    w1ts = [w1t0, w1t1, w1t2, w1t3]
    w2ts = [w2t0, w2t1, w2t2, w2t3]
    outs = []
    for i in range(4):
        k = 8 * (i + 1)
        a = feats[:k, :]
        h = jnp.maximum(
            jnp.dot(w0ts[i][...], a, preferred_element_type=jnp.float32,
                    precision=lax.Precision.HIGHEST), 0.0)
        h = jnp.maximum(
            jnp.dot(w1ts[i][...], h, preferred_element_type=jnp.float32,
                    precision=lax.Precision.HIGHEST), 0.0)
        outs.append(
            jnp.dot(w2ts[i][...], h, preferred_element_type=jnp.float32,
                    precision=lax.Precision.HIGHEST))
    out_ref[...] = jnp.concatenate(outs, axis=0)


def _mlp(pos_t, g3, w0ts, w1ts, w2ts):
    b = pos_t.shape[1]
    m = b // _BP
    res = jnp.asarray(_RES_F32, jnp.float32).reshape(16, 1)
    full = lambda arr: pl.BlockSpec(arr.shape, lambda i: (0, 0))
    in_specs = [
        pl.BlockSpec((16, 1), lambda i: (0, 0)),
        pl.BlockSpec((2, _BP), lambda i: (0, i)),
        pl.BlockSpec((64, _BP), lambda i: (i, 0)),
    ] + [full(w) for w in w0ts + w1ts + w2ts]
    return pl.pallas_call(
        _mlp_body,
        grid=(m,),
        in_specs=in_specs,
        out_specs=pl.BlockSpec((4, _BP), lambda i: (0, i)),
        out_shape=jax.ShapeDtypeStruct((4, b), jnp.float32),
    )(res, pos_t, g3, *w0ts, *w1ts, *w2ts)


def _prep_weights(mlp_params):
    w0ts, w1ts, w2ts = [], [], []
    for i in range(4):
        w0, w1, w2 = mlp_params[i]
        prows = []
        for j in range(i + 1):
            prows.extend(8 * j + m for m in _ROW_PERM)
        w0ts.append(w0[np.array(prows), :].T)
        w1ts.append(w1.T)
        w2ts.append(w2.T)
    return w0ts, w1ts, w2ts


def kernel(v_pixel_pos, hash_table, mlp_params):
    b = v_pixel_pos.shape[0]
    pos_t = v_pixel_pos.T
    idx2 = _compute_idx(pos_t)
    table_i32 = lax.bitcast_convert_type(
        hash_table.astype(jnp.bfloat16), jnp.int32).reshape(_NL * _T)
    g3 = _sc_gather(table_i32, idx2)
    w0ts, w1ts, w2ts = _prep_weights(mlp_params)
    out4 = _mlp(pos_t, g3, w0ts, w1ts, w2ts)
    return tuple(out4[i].reshape(b, 1) for i in range(4))


# speedup vs baseline: 1.1825x; 1.0187x over previous
import functools

import jax
import jax.numpy as jnp
import numpy as np
from jax import lax
from jax.experimental import pallas as pl
from jax.experimental.pallas import tpu as pltpu
from jax.experimental.pallas import tpu_sc as plsc

_NL = 16
_T = 1 << 19
_PRIME_I32 = -1640531535
_MASK = _T - 1
_LVL = [0, 1, 8, 9, 2, 3, 10, 11, 4, 5, 12, 13, 6, 7, 14, 15]
_ROW_PERM = [0, 2, 4, 6, 1, 3, 5, 7]

_BP = 2048
_GR = 16

_RES_F32 = [float(1 << (4 + lvl)) for lvl in _LVL]
_BASE_I32 = [lvl * _T for lvl in _LVL]


def _idx_body(res_ref, base_ref, pos_ref, out_ref):
    px = pos_ref[0:1, :]
    py = pos_ref[1:2, :]
    res = res_ref[...]
    base = base_ref[...]
    sx = res * px
    sy = res * py
    x0 = jnp.floor(sx).astype(jnp.int32)
    y0 = jnp.floor(sy).astype(jnp.int32)
    rows = []
    for dx in (0, 1):
        cx = x0 + jnp.int32(dx)
        for dy in (0, 1):
            cy = y0 + jnp.int32(dy)
            h = (cx ^ (cy * jnp.int32(_PRIME_I32))) & jnp.int32(_MASK)
            rows.append(h + base)
    out_ref[...] = jnp.concatenate(rows, axis=0)


def _compute_idx(pos_t):
    b = pos_t.shape[1]
    m = b // _BP
    res = jnp.asarray(_RES_F32, jnp.float32).reshape(16, 1)
    base = jnp.asarray(_BASE_I32, jnp.int32).reshape(16, 1)
    return pl.pallas_call(
        _idx_body,
        grid=(m,),
        in_specs=[
            pl.BlockSpec((16, 1), lambda i: (0, 0)),
            pl.BlockSpec((16, 1), lambda i: (0, 0)),
            pl.BlockSpec((2, _BP), lambda i: (0, i)),
        ],
        out_specs=pl.BlockSpec((64, _BP), lambda i: (i, 0)),
        out_shape=jax.ShapeDtypeStruct((m * 64, _BP), jnp.int32),
    )(res, base, pos_t)


def _sc_gather(table_i32, idx2):
    nrows = idx2.shape[0]
    mesh = plsc.VectorSubcoreMesh(
        core_axis_name="core", subcore_axis_name="subcore")

    @functools.partial(
        pl.kernel,
        out_type=jax.ShapeDtypeStruct((nrows, _BP), jnp.int32),
        mesh=mesh,
        scratch_types=[pltpu.SemaphoreType.DMA],
        compiler_params=pltpu.CompilerParams(use_tc_tiling_on_sc=False),
    )
    def k(table_hbm, idx_hbm, out_hbm, sem):
        def body(i_vmem, o_vmem):
            handles = [
                pltpu.async_copy(
                    table_hbm.at[i_vmem.at[j]], o_vmem.at[j], sem)
                for j in range(_GR)
            ]
            for h in handles:
                h.wait()

        pltpu.emit_pipeline(
            body,
            grid=(nrows // _GR,),
            in_specs=[pl.BlockSpec((_GR, _BP), lambda i: (i, 0))],
            out_specs=[pl.BlockSpec((_GR, _BP), lambda i: (i, 0))],
            core_axis_name=("core", "subcore"),
            dimension_semantics=(pltpu.PARALLEL,),
        )(idx_hbm, out_hbm)

    return k(table_i32, idx2)


def _unpack_lo(x):
    return lax.bitcast_convert_type(jnp.left_shift(x, 16), jnp.float32)


def _unpack_hi(x):
    return lax.bitcast_convert_type(
        jnp.bitwise_and(x, jnp.int32(-65536)), jnp.float32)


def _mlp_body(res_ref, pos_ref, g_ref,
              w0t0, w0t1, w0t2, w0t3,
              w1t0, w1t1, w1t2, w1t3,
              w2t0, w2t1, w2t2, w2t3,
              out_ref):
    px = pos_ref[0:1, :]
    py = pos_ref[1:2, :]
    res = res_ref[...]
    sx = res * px
    sy = res * py
    wx1 = sx - jnp.floor(sx)
    wy1 = sy - jnp.floor(sy)
    wx0 = 1.0 - wx1
    wy0 = 1.0 - wy1
    wc = (wx0 * wy0, wx0 * wy1, wx1 * wy0, wx1 * wy1)
    f0 = None
    f1 = None
    for c in range(4):
        p = g_ref[16 * c:16 * c + 16, :]
        a = _unpack_lo(p) * wc[c]
        b = _unpack_hi(p) * wc[c]
        f0 = a if f0 is None else f0 + a
        f1 = b if f1 is None else f1 + b
    parts = []
    for i in range(4):
        parts.append(f0[4 * i:4 * i + 4, :])
        parts.append(f1[4 * i:4 * i + 4, :])
    feats = jnp.concatenate(parts, axis=0)
    w0ts = [w0t0, w0t1, w0t2, w0t3]
    w1ts = [w1t0, w1t1, w1t2, w1t3]
    w2ts = [w2t0, w2t1, w2t2, w2t3]
    outs = []
    for i in range(4):
        k = 8 * (i + 1)
        a = feats[:k, :]
        h = jnp.maximum(
            jnp.dot(w0ts[i][...], a, preferred_element_type=jnp.float32,
                    precision=lax.Precision.HIGHEST), 0.0)
        h = jnp.maximum(
            jnp.dot(w1ts[i][...], h, preferred_element_type=jnp.float32,
                    precision=lax.Precision.HIGHEST), 0.0)
        outs.append(
            jnp.dot(w2ts[i][...], h, preferred_element_type=jnp.float32,
                    precision=lax.Precision.HIGHEST))
    out_ref[...] = jnp.concatenate(outs, axis=0)


def _mlp(pos_t, g3, w0ts, w1ts, w2ts):
    b = pos_t.shape[1]
    m = b // _BP
    res = jnp.asarray(_RES_F32, jnp.float32).reshape(16, 1)
    full = lambda arr: pl.BlockSpec(arr.shape, lambda i: (0, 0))
    in_specs = [
        pl.BlockSpec((16, 1), lambda i: (0, 0)),
        pl.BlockSpec((2, _BP), lambda i: (0, i)),
        pl.BlockSpec((64, _BP), lambda i: (i, 0)),
    ] + [full(w) for w in w0ts + w1ts + w2ts]
    return pl.pallas_call(
        _mlp_body,
        grid=(m,),
        in_specs=in_specs,
        out_specs=pl.BlockSpec((4, _BP), lambda i: (0, i)),
        out_shape=jax.ShapeDtypeStruct((4, b), jnp.float32),
    )(res, pos_t, g3, *w0ts, *w1ts, *w2ts)


def _prep_weights(mlp_params):
    w0ts, w1ts, w2ts = [], [], []
    for i in range(4):
        w0, w1, w2 = mlp_params[i]
        prows = []
        for j in range(i + 1):
            prows.extend(8 * j + m for m in _ROW_PERM)
        w0ts.append(w0[np.array(prows), :].T)
        w1ts.append(w1.T)
        w2ts.append(w2.T)
    return w0ts, w1ts, w2ts


def kernel(v_pixel_pos, hash_table, mlp_params):
    b = v_pixel_pos.shape[0]
    pos_t = v_pixel_pos.T
    table_i32 = lax.bitcast_convert_type(
        hash_table.astype(jnp.bfloat16), jnp.int32).reshape(_NL * _T)
    w0ts, w1ts, w2ts = _prep_weights(mlp_params)
    nc = 2
    bc = b // nc
    outs = []
    for c in range(nc):
        p = lax.slice(pos_t, (0, c * bc), (2, (c + 1) * bc))
        idx2 = _compute_idx(p)
        g = _sc_gather(table_i32, idx2)
        outs.append(_mlp(p, g, w0ts, w1ts, w2ts))
    out4 = jnp.concatenate(outs, axis=1)
    return tuple(out4[i].reshape(b, 1) for i in range(4))
